# baseline (speedup 1.0000x reference)
import functools

import jax
import jax.numpy as jnp
from jax import lax
from jax.experimental import pallas as pl
from jax.experimental.pallas import tpu as pltpu
from jax.experimental.pallas import tpu_sc as plsc

_NC = 2
_NS = 16
_NW = _NC * _NS
_L = 16

_SELU_SCALE = 1.0507009873554805
_SELU_ALPHA = 1.6732632423543772


def _sc_partials(x, src, dst, w, n, d):
    e = src.size
    b = 128
    sbb = 8
    epw = -(-e // (_NW * b * sbb)) * b * sbb
    ep = epw * _NW
    pad = ep - e
    nb = epw // b
    nsb = nb // sbb

    idx_pad = jnp.arange(pad, dtype=jnp.int32) % n
    src_p = jnp.concatenate([src, idx_pad]).reshape(_NW, nb, b)
    dst_p = jnp.concatenate([dst, idx_pad]).reshape(_NW, nb, b)
    w_p = jnp.concatenate([w, jnp.zeros((pad,), jnp.float32)]).reshape(_NW, nb, b)

    npad = -(-n // (_NS * 128)) * _NS * 128
    rpt = npad // _NS
    zb = 128
    nz = rpt // zb

    mesh = plsc.VectorSubcoreMesh(core_axis_name="c", subcore_axis_name="s")

    @functools.partial(
        pl.kernel,
        mesh=mesh,
        compiler_params=pltpu.CompilerParams(use_tc_tiling_on_sc=False),
        out_type=jax.ShapeDtypeStruct((_NC, npad, d), jnp.float32),
        scratch_types=[
            pltpu.VMEM((1, sbb, b), jnp.int32),
            pltpu.VMEM((1, sbb, b), jnp.int32),
            pltpu.VMEM((1, sbb, b), jnp.float32),
            pltpu.VMEM((1, b, d // 2), jnp.uint32),
            pltpu.VMEM((1, b, d), jnp.float32),
            pltpu.VMEM_SHARED((npad, d), jnp.float32),
            pltpu.SemaphoreType.DMA,
            pltpu.SemaphoreType.DMA,
            pltpu.SemaphoreType.DMA,
        ],
    )
    def scatter_kernel(x_hbm, src_hbm, dst_hbm, w_hbm, out_hbm,
                       src_v, dst_v, w_v, rows_v, sc_v, agg_sh, gsem, ssem, stsem):
        cid = lax.axis_index("c")
        sid = lax.axis_index("s")
        wid = sid * _NC + cid

        def gwait(p, st, j):
            pltpu.make_async_copy(x_hbm.at[src_v.at[st].at[j]],
                                  rows_v.at[p], gsem).wait()

        def swait(p, st, j):
            pltpu.make_async_copy(rows_v.at[p],
                                  agg_sh.at[dst_v.at[st].at[j]], ssem).wait()

        def stage(g, q):
            grp = pl.ds(g * sbb, sbb)
            pltpu.async_copy(src_hbm.at[wid].at[grp], src_v.at[q], stsem)
            pltpu.async_copy(dst_hbm.at[wid].at[grp], dst_v.at[q], stsem)
            pltpu.async_copy(w_hbm.at[wid].at[grp], w_v.at[q], stsem)

        def stwait(q):
            grp = pl.ds(0, sbb)
            pltpu.make_async_copy(src_hbm.at[wid].at[grp], src_v.at[q], stsem).wait()
            pltpu.make_async_copy(dst_hbm.at[wid].at[grp], dst_v.at[q], stsem).wait()
            pltpu.make_async_copy(w_hbm.at[wid].at[grp], w_v.at[q], stsem).wait()

        def zstore(i, carry):
            for q in range(d // _L):
                sc_v[0, i, pl.ds(q * _L, _L)] = jnp.zeros((_L,), jnp.float32)
            return carry
        lax.fori_loop(0, b, zstore, 0)
        for q in range(nz * zb // b):
            pltpu.sync_copy(sc_v.at[0], agg_sh.at[pl.ds(sid * rpt + q * b, b)])
        plsc.subcore_barrier()

        def super_block(sb, carry):
            grp = pl.ds(sb * sbb, sbb)
            pltpu.sync_copy(src_hbm.at[wid].at[grp], src_v.at[0])
            pltpu.sync_copy(dst_hbm.at[wid].at[grp], dst_v.at[0])
            pltpu.sync_copy(w_hbm.at[wid].at[grp], w_v.at[0])

            def block(j, bcarry):
                pltpu.async_copy(x_hbm.at[src_v.at[0].at[j]],
                                 rows_v.at[0], gsem).wait()

                def row16(i16, rcarry):
                    wv = w_v[0, j, pl.ds(i16 * _L, _L)]
                    for k in range(_L):
                        ws = jnp.full((_L,), wv[k], jnp.float32)
                        r = i16 * _L + k
                        for q in range(d // (2 * _L)):
                            word = rows_v[0, r, pl.ds(q * _L, _L)]
                            va = lax.bitcast_convert_type(word << 16, jnp.float32)
                            vb = lax.bitcast_convert_type(
                                word & jnp.uint32(0xFFFF0000), jnp.float32)
                            sc_v[0, r, pl.ds(q * 2 * _L, _L)] = va * ws
                            sc_v[0, r, pl.ds(q * 2 * _L + _L, _L)] = vb * ws
                    return rcarry
                lax.fori_loop(0, b // _L, row16, 0)

                pltpu.async_copy(sc_v.at[0], agg_sh.at[dst_v.at[0].at[j]],
                                 ssem, add=True).wait()
                return bcarry
            lax.fori_loop(0, sbb, block, 0)
            return carry
        lax.fori_loop(0, nsb, super_block, 0)

        plsc.subcore_barrier()
        for q in range(nz * zb // b):
            rows = pl.ds(sid * rpt + q * b, b)
            pltpu.sync_copy(agg_sh.at[rows], out_hbm.at[cid].at[rows])

    return scatter_kernel(x, src_p, dst_p, w_p)


def _epilogue(x, partials, kmat, kperm, bias2, skip2, n, d, c):
    rb = 1000

    def body(x_ref, p_ref, k_ref, kp_ref, b_ref, s_ref, o_ref):
        acc = jnp.dot(x_ref[...], k_ref[...] * s_ref[...],
                      preferred_element_type=jnp.float32)
        acc = acc + jnp.dot(p_ref[0] + p_ref[1], kp_ref[...],
                            preferred_element_type=jnp.float32)
        acc = acc + b_ref[...]
        neg = _SELU_ALPHA * (jnp.exp(jnp.minimum(acc, 0.0)) - 1.0)
        o_ref[...] = _SELU_SCALE * jnp.where(acc > 0.0, acc, neg)

    return pl.pallas_call(
        body,
        grid=(n // rb,),
        in_specs=[
            pl.BlockSpec((rb, d), lambda i: (i, 0)),
            pl.BlockSpec((_NC, rb, c), lambda i: (0, i, 0)),
            pl.BlockSpec((d, c), lambda i: (0, 0)),
            pl.BlockSpec((d, c), lambda i: (0, 0)),
            pl.BlockSpec((1, c), lambda i: (0, 0)),
            pl.BlockSpec((1, c), lambda i: (0, 0)),
        ],
        out_specs=pl.BlockSpec((rb, c), lambda i: (i, 0)),
        out_shape=jax.ShapeDtypeStruct((n, c), jnp.float32),
    )(x, partials, kmat, kperm, bias2, skip2)


def kernel(features, edge_index, edge_weight, kernel, bias, skip_weight):
    n, d = features.shape
    c = kernel.shape[1]
    dst = edge_index[0]
    src = edge_index[1]
    xw = lax.bitcast_convert_type(
        features.astype(jnp.bfloat16).reshape(n, d // 2, 2), jnp.uint32)
    partials = _sc_partials(xw, src, dst, edge_weight, n, d)
    perm = [q * 32 + (2 * t if t < 16 else 2 * (t - 16) + 1)
            for q in range(d // 32) for t in range(32)]
    kperm = kernel[jnp.array(perm, dtype=jnp.int32)]
    return _epilogue(features, partials, kernel, kperm,
                     bias.reshape(1, c), skip_weight.reshape(1, c), n, d, c)


# speedup vs baseline: 1.9701x; 1.9701x over previous
import functools

import jax
import jax.numpy as jnp
from jax import lax
from jax.experimental import pallas as pl
from jax.experimental.pallas import tpu as pltpu
from jax.experimental.pallas import tpu_sc as plsc

_NC = 2
_NS = 16
_NW = _NC * _NS
_L = 16

_SELU_SCALE = 1.0507009873554805
_SELU_ALPHA = 1.6732632423543772


def _sc_partials(x, src, dst, w, n, d):
    e = src.size
    b = 128
    sbb = 8
    epw = -(-e // (_NW * b * sbb)) * b * sbb
    ep = epw * _NW
    pad = ep - e
    nb = epw // b
    nsb = nb // sbb

    idx_pad = jnp.arange(pad, dtype=jnp.int32) % n
    src_p = jnp.concatenate([src, idx_pad]).reshape(_NW, nb, b)
    dst_p = jnp.concatenate([dst, idx_pad]).reshape(_NW, nb, b)
    w_p = jnp.concatenate([w, jnp.zeros((pad,), jnp.float32)]).reshape(_NW, nb, b)

    npad = -(-n // (_NS * 128)) * _NS * 128
    rpt = npad // _NS
    zb = 128
    nz = rpt // zb

    mesh = plsc.VectorSubcoreMesh(core_axis_name="c", subcore_axis_name="s")

    @functools.partial(
        pl.kernel,
        mesh=mesh,
        out_type=jax.ShapeDtypeStruct((_NC, npad, d), jnp.float32),
        scratch_types=[
            pltpu.VMEM((1, sbb, b), jnp.int32),
            pltpu.VMEM((1, sbb, b), jnp.int32),
            pltpu.VMEM((1, sbb, b), jnp.float32),
            pltpu.VMEM((2, b, d), jnp.float32),
            pltpu.VMEM_SHARED((npad, d), jnp.float32),
            pltpu.SemaphoreType.DMA,
            pltpu.SemaphoreType.DMA,
            pltpu.SemaphoreType.DMA,
            pltpu.SemaphoreType.DMA,
            pltpu.SemaphoreType.DMA,
        ],
    )
    def scatter_kernel(x_hbm, src_hbm, dst_hbm, w_hbm, out_hbm,
                       src_v, dst_v, w_v, rows_v, agg_sh,
                       gsem0, gsem1, ssem0, ssem1, stsem):
        cid = lax.axis_index("c")
        sid = lax.axis_index("s")
        wid = sid * _NC + cid

        def gwait(p, st, j):
            pltpu.make_async_copy(x_hbm.at[src_v.at[st].at[j]],
                                  rows_v.at[p], gsem).wait()

        def swait(p, st, j):
            pltpu.make_async_copy(rows_v.at[p],
                                  agg_sh.at[dst_v.at[st].at[j]], ssem).wait()

        def stage(g, q):
            grp = pl.ds(g * sbb, sbb)
            pltpu.async_copy(src_hbm.at[wid].at[grp], src_v.at[q], stsem)
            pltpu.async_copy(dst_hbm.at[wid].at[grp], dst_v.at[q], stsem)
            pltpu.async_copy(w_hbm.at[wid].at[grp], w_v.at[q], stsem)

        def stwait(q):
            grp = pl.ds(0, sbb)
            pltpu.make_async_copy(src_hbm.at[wid].at[grp], src_v.at[q], stsem).wait()
            pltpu.make_async_copy(dst_hbm.at[wid].at[grp], dst_v.at[q], stsem).wait()
            pltpu.make_async_copy(w_hbm.at[wid].at[grp], w_v.at[q], stsem).wait()

        def zstore(i, carry):
            for q in range(d // _L):
                rows_v[0, i, pl.ds(q * _L, _L)] = jnp.zeros((_L,), jnp.float32)
            return carry
        lax.fori_loop(0, b, zstore, 0)
        for q in range(nz * zb // b):
            pltpu.sync_copy(rows_v.at[0], agg_sh.at[pl.ds(sid * rpt + q * b, b)])
        plsc.subcore_barrier()

        def super_block(sb, carry):
            grp = pl.ds(sb * sbb, sbb)
            pltpu.sync_copy(src_hbm.at[wid].at[grp], src_v.at[0])
            pltpu.sync_copy(dst_hbm.at[wid].at[grp], dst_v.at[0])
            pltpu.sync_copy(w_hbm.at[wid].at[grp], w_v.at[0])

            def scale(p, j):
                def row16(i16, rcarry):
                    wv = w_v[0, j, pl.ds(i16 * _L, _L)]
                    for k in range(_L):
                        ws = jnp.full((_L,), wv[k], jnp.float32)
                        r = i16 * _L + k
                        for q in range(d // _L):
                            rows_v[p, r, pl.ds(q * _L, _L)] = (
                                rows_v[p, r, pl.ds(q * _L, _L)] * ws)
                    return rcarry
                lax.fori_loop(0, b // _L, row16, 0)

            def block2(jj, bcarry):
                j0 = 2 * jj
                j1 = 2 * jj + 1
                h0 = pltpu.async_copy(x_hbm.at[src_v.at[0].at[j0]],
                                     rows_v.at[0], gsem0)
                h1 = pltpu.async_copy(x_hbm.at[src_v.at[0].at[j1]],
                                     rows_v.at[1], gsem1)
                h0.wait()
                scale(0, j0)
                s0 = pltpu.async_copy(rows_v.at[0], agg_sh.at[dst_v.at[0].at[j0]],
                                      ssem0, add=True)
                h1.wait()
                scale(1, j1)
                s1 = pltpu.async_copy(rows_v.at[1], agg_sh.at[dst_v.at[0].at[j1]],
                                      ssem1, add=True)
                s0.wait()
                s1.wait()
                return bcarry
            lax.fori_loop(0, sbb // 2, block2, 0)
            return carry
        lax.fori_loop(0, nsb, super_block, 0)

        plsc.subcore_barrier()
        for q in range(nz * zb // b):
            rows = pl.ds(sid * rpt + q * b, b)
            pltpu.sync_copy(agg_sh.at[rows], out_hbm.at[cid].at[rows])

    return scatter_kernel(x, src_p, dst_p, w_p)


def _epilogue(x, partials, kmat, bias2, skip2, n, d, c):
    rb = 1000

    def body(x_ref, p_ref, k_ref, b_ref, s_ref, o_ref):
        km = k_ref[...]
        acc = jnp.dot(x_ref[...], km * s_ref[...], preferred_element_type=jnp.float32)
        acc = acc + jnp.dot(p_ref[0] + p_ref[1], km, preferred_element_type=jnp.float32)
        acc = acc + b_ref[...]
        neg = _SELU_ALPHA * (jnp.exp(jnp.minimum(acc, 0.0)) - 1.0)
        o_ref[...] = _SELU_SCALE * jnp.where(acc > 0.0, acc, neg)

    return pl.pallas_call(
        body,
        grid=(n // rb,),
        in_specs=[
            pl.BlockSpec((rb, d), lambda i: (i, 0)),
            pl.BlockSpec((_NC, rb, c), lambda i: (0, i, 0)),
            pl.BlockSpec((d, c), lambda i: (0, 0)),
            pl.BlockSpec((1, c), lambda i: (0, 0)),
            pl.BlockSpec((1, c), lambda i: (0, 0)),
        ],
        out_specs=pl.BlockSpec((rb, c), lambda i: (i, 0)),
        out_shape=jax.ShapeDtypeStruct((n, c), jnp.float32),
    )(x, partials, kmat, bias2, skip2)


def kernel(features, edge_index, edge_weight, kernel, bias, skip_weight):
    n, d = features.shape
    c = kernel.shape[1]
    dst = edge_index[0]
    src = edge_index[1]
    partials = _sc_partials(features, src, dst, edge_weight, n, d)
    return _epilogue(features, partials, kernel,
                     bias.reshape(1, c), skip_weight.reshape(1, c), n, d, c)
